# baseline (speedup 1.0000x reference)
import jax
import jax.numpy as jnp
from jax.experimental import pallas as pl

_BB = 8


def _body(x_ref, o_ref):
    x = x_ref[...]
    best = x[:, 0, :]
    bidx = jnp.zeros(best.shape, jnp.int32)
    for i in range(1, x.shape[1]):
        v = x[:, i, :]
        m = v < best
        best = jnp.where(m, v, best)
        bidx = jnp.where(m, jnp.int32(i), bidx)
    o_ref[...] = bidx


def kernel(x):
    B, R, C = x.shape
    return pl.pallas_call(
        _body,
        grid=(B // _BB,),
        in_specs=[pl.BlockSpec((_BB, R, C), lambda i: (i, 0, 0))],
        out_specs=pl.BlockSpec((_BB, C), lambda i: (i, 0)),
        out_shape=jax.ShapeDtypeStruct((B, C), jnp.int32),
    )(x)


# speedup vs baseline: 3.3053x; 3.3053x over previous
import jax
import jax.numpy as jnp
from jax.experimental import pallas as pl
from jax.experimental.pallas import tpu as pltpu

_BB = 8


def _body(x_ref, o_ref):
    x = x_ref[...]
    C = x.shape[2]
    iota_s = jax.lax.broadcasted_iota(jnp.int32, (8, C), 0)
    out = jnp.zeros((8, C), jnp.int32)
    for b in range(_BB):
        xb = x[b]
        g = [xb[8 * k:8 * (k + 1), :] for k in range(4)]
        v = jnp.minimum(jnp.minimum(g[0], g[1]), jnp.minimum(g[2], g[3]))
        for sh in (4, 2, 1):
            v = jnp.minimum(v, pltpu.roll(v, sh, axis=0))
        idx = jnp.full((8, C), 64, jnp.int32)
        for k in range(4):
            idx = jnp.minimum(idx, jnp.where(g[k] == v, iota_s + 8 * k, 64))
        for sh in (4, 2, 1):
            idx = jnp.minimum(idx, pltpu.roll(idx, sh, axis=0))
        out = jnp.where(iota_s == b, idx, out)
    o_ref[...] = out


def kernel(x):
    B, R, C = x.shape
    return pl.pallas_call(
        _body,
        grid=(B // _BB,),
        in_specs=[pl.BlockSpec((_BB, R, C), lambda i: (i, 0, 0))],
        out_specs=pl.BlockSpec((_BB, C), lambda i: (i, 0)),
        out_shape=jax.ShapeDtypeStruct((B, C), jnp.int32),
    )(x)
